# baseline (speedup 1.0000x reference)
import jax
import jax.numpy as jnp
from jax import lax
from jax.experimental import pallas as pl
from jax.experimental.pallas import tpu as pltpu

_N = 1024
_K = 10
_H = 5


def _fused_kernel(sn_ref, tn_ref, ssc_ref, ssr_ref, ttc_ref, ttr_ref,
                  idxc_ref, idxr_ref, loss_ref,
                  wp, keys, wnn, ds, mbuf, wchat):
    f32 = jnp.float32
    bf16 = jnp.bfloat16
    i32 = jnp.int32
    iota_row = lax.broadcasted_iota(i32, (_N, _N), 1)
    iota_col = lax.broadcasted_iota(i32, (_N, _N), 0)
    eye = iota_col == iota_row

    tn = tn_ref[...]
    gt = lax.dot_general(tn, tn, (((1,), (1,)), ((), ())),
                         preferred_element_type=f32)
    d2t = jnp.maximum(ttc_ref[...] + ttr_ref[...] - 2.0 * gt, 0.0)
    wp[...] = 0.5 * jnp.exp(-d2t)
    same = idxc_ref[...] == idxr_ref[...]

    bits = lax.bitcast_convert_type(jnp.where(same, 4.0, 4.0 - d2t), i32)
    packed = (bits & i32(~1023)) | (i32(1023) - iota_row)
    keys[...] = lax.bitcast_convert_type(packed, jnp.float32)

    sn = sn_ref[...]
    gs = lax.dot_general(sn, sn, (((1,), (1,)), ((), ())),
                         preferred_element_type=f32)
    d2s = ssc_ref[...] + ssr_ref[...] - 2.0 * gs
    ds[...] = jnp.sqrt(jnp.maximum(d2s, 0.0))

    m = jnp.max(keys[...], axis=1, keepdims=True)
    m_half = m
    for k in range(1, _K):
        kv = keys[...]
        m = jnp.max(jnp.where(kv < m, kv, -jnp.inf), axis=1, keepdims=True)
        if k == _H - 1:
            m_half = m

    kv = keys[...]
    wnn[...] = jnp.where(kv >= m, 1.0, 0.0).astype(bf16)
    gh_bf = jnp.where(kv >= m_half, 0.25 / _H, 0.0).astype(bf16)

    v_bf = wnn[...] * wnn[...].T
    v = v_bf.astype(f32)
    denom = jnp.sum(v, axis=1, keepdims=True)
    inv_denom = 1.0 / jnp.where(denom > 0, denom, 1.0)
    m_mat = lax.dot_general(v_bf, v_bf, (((1,), (1,)), ((), ())),
                            preferred_element_type=f32)
    mbuf[...] = v * m_mat * inv_denom

    wchat[...] = lax.dot_general(gh_bf, mbuf[...].astype(bf16),
                                 (((1,), (0,)), ((), ())),
                                 preferred_element_type=f32)
    wchat_t = wchat[...].T

    d = ds[...]
    inv_mu = float(_N) / jnp.sum(d, axis=1, keepdims=True)
    s = d * inv_mu
    a = s * s
    r = jnp.maximum(1.0 - s, 0.0)
    b = r * r
    w_full = wp[...] + wchat[...] + wchat_t
    term = b + (a - b) * w_full
    term = jnp.where(eye, 0.0, term)
    total = jnp.sum(jnp.sum(term, axis=1, keepdims=True), axis=0, keepdims=True)
    loss_ref[...] = total / (_N * (_N - 1))


def _run(sn, tn, ssc, ssr, ttc, ttr, idxc, idxr, interpret=False):
    scr = [pltpu.VMEM((_N, _N), jnp.float32),
           pltpu.VMEM((_N, _N), jnp.float32),
           pltpu.VMEM((_N, _N), jnp.bfloat16),
           pltpu.VMEM((_N, _N), jnp.float32),
           pltpu.VMEM((_N, _N), jnp.float32),
           pltpu.VMEM((_N, _N), jnp.float32)]
    return pl.pallas_call(
        _fused_kernel,
        out_shape=jax.ShapeDtypeStruct((1, 1), jnp.float32),
        scratch_shapes=scr,
        interpret=interpret,
    )(sn, tn, ssc, ssr, ttc, ttr, idxc, idxr)


def kernel(s_emb, t_emb, idx):
    def _norm(x):
        n = jnp.sqrt(jnp.sum(x * x, axis=1, keepdims=True))
        return x / jnp.maximum(n, 1e-12)

    sn = _norm(s_emb)
    tn = _norm(t_emb)
    ss = jnp.sum(sn * sn, axis=1)
    tt = jnp.sum(tn * tn, axis=1)
    idx32 = idx.astype(jnp.int32)
    out = _run(sn, tn,
               ss[:, None], ss[None, :],
               tt[:, None], tt[None, :],
               idx32[:, None], idx32[None, :])
    return out[0, 0]


# speedup vs baseline: 1.3300x; 1.3300x over previous
import jax
import jax.numpy as jnp
from jax import lax
from jax.experimental import pallas as pl
from jax.experimental.pallas import tpu as pltpu

_N = 1024
_K = 10
_H = 5


def _fused_kernel(s_ref, t_ref, idxc_ref, idxr_ref, loss_ref,
                  wp, keys, wnn, ds, mbuf, wchat):
    f32 = jnp.float32
    bf16 = jnp.bfloat16
    i32 = jnp.int32
    iota_row = lax.broadcasted_iota(i32, (_N, _N), 1)
    iota_col = lax.broadcasted_iota(i32, (_N, _N), 0)
    eye = iota_col == iota_row

    def _norm(x):
        n = jnp.sqrt(jnp.sum(x * x, axis=1, keepdims=True))
        return x / jnp.maximum(n, 1e-12)

    tn = _norm(t_ref[...])
    gt = lax.dot_general(tn, tn, (((1,), (1,)), ((), ())),
                         preferred_element_type=f32)
    d2t = jnp.maximum(2.0 - 2.0 * gt, 0.0)
    wp[...] = 0.5 * jnp.exp(-d2t)
    same = idxc_ref[...] == idxr_ref[...]

    bits = lax.bitcast_convert_type(jnp.where(same, 4.0, 4.0 - d2t), i32)
    packed = (bits & i32(~1023)) | (i32(1023) - iota_row)
    keys[...] = lax.bitcast_convert_type(packed, jnp.float32)

    sn = _norm(s_ref[...])
    gs = lax.dot_general(sn, sn, (((1,), (1,)), ((), ())),
                         preferred_element_type=f32)
    ds[...] = jnp.sqrt(jnp.maximum(2.0 - 2.0 * gs, 0.0))

    m = jnp.max(keys[...], axis=1, keepdims=True)
    m_half = m
    for k in range(1, _K):
        kv = keys[...]
        m = jnp.max(jnp.where(kv < m, kv, -jnp.inf), axis=1, keepdims=True)
        if k == _H - 1:
            m_half = m

    kv = keys[...]
    wnn[...] = jnp.where(kv >= m, 1.0, 0.0).astype(bf16)
    gh_bf = jnp.where(kv >= m_half, 0.25 / _H, 0.0).astype(bf16)

    v_bf = wnn[...] * wnn[...].T
    v = v_bf.astype(f32)
    denom = jnp.sum(v, axis=1, keepdims=True)
    inv_denom = 1.0 / jnp.where(denom > 0, denom, 1.0)
    m_mat = lax.dot_general(v_bf, v_bf, (((1,), (1,)), ((), ())),
                            preferred_element_type=f32)
    mbuf[...] = v * m_mat * inv_denom

    wchat[...] = lax.dot_general(gh_bf, mbuf[...].astype(bf16),
                                 (((1,), (0,)), ((), ())),
                                 preferred_element_type=f32)
    wchat_t = wchat[...].T

    d = ds[...]
    inv_mu = float(_N) / jnp.sum(d, axis=1, keepdims=True)
    s = d * inv_mu
    a = s * s
    r = jnp.maximum(1.0 - s, 0.0)
    b = r * r
    w_full = wp[...] + wchat[...] + wchat_t
    term = b + (a - b) * w_full
    term = jnp.where(eye, 0.0, term)
    total = jnp.sum(jnp.sum(term, axis=1, keepdims=True), axis=0, keepdims=True)
    loss_ref[...] = total / (_N * (_N - 1))


def _run(s_emb, t_emb, idxc, idxr, interpret=False):
    scr = [pltpu.VMEM((_N, _N), jnp.float32),
           pltpu.VMEM((_N, _N), jnp.float32),
           pltpu.VMEM((_N, _N), jnp.bfloat16),
           pltpu.VMEM((_N, _N), jnp.float32),
           pltpu.VMEM((_N, _N), jnp.float32),
           pltpu.VMEM((_N, _N), jnp.float32)]
    return pl.pallas_call(
        _fused_kernel,
        out_shape=jax.ShapeDtypeStruct((1, 1), jnp.float32),
        scratch_shapes=scr,
        interpret=interpret,
    )(s_emb, t_emb, idxc, idxr)


def kernel(s_emb, t_emb, idx):
    idx32 = idx.astype(jnp.int32)
    out = _run(s_emb, t_emb, idx32[:, None], idx32[None, :])
    return out[0, 0]
